# baseline (speedup 1.0000x reference)
import math

import numpy as np
import jax
import jax.numpy as jnp
from jax import lax
from jax.experimental import pallas as pl
from jax.experimental.pallas import tpu as pltpu
from jax.experimental.pallas import tpu_sc as plsc

_NUM_EMB = 1000000
_DIM = 64
_BATCH = 4096
_SEQ = 200
_LANES = 16

_NC, _NS = 2, 16
_NW = _NC * _NS
_SEQ_PER_W = _BATCH // _NW
_CHUNK_SEQ = 2
_ROWS = _CHUNK_SEQ * _SEQ
_N_CHUNKS = _SEQ_PER_W // _CHUNK_SEQ


def _pos_encoding():
    pos = np.arange(_SEQ, dtype=np.float32)[:, None]
    div = np.exp(np.arange(0, _DIM, 2, dtype=np.float32)
                 * -(math.log(10000.0) / _DIM))
    pe = np.zeros((_SEQ, _DIM), dtype=np.float32)
    pe[:, 0::2] = np.sin(pos * div)
    pe[:, 1::2] = np.cos(pos * div)
    return np.tile(pe, (_CHUNK_SEQ, 1))


_PE = _pos_encoding()


def _body(x_hbm, pe_hbm, table_hbm, out_hbm, idx_v, rows_v, pe_v, sem):
    wid = lax.axis_index("s") * _NC + lax.axis_index("c")
    base = wid * _SEQ_PER_W * _SEQ
    pltpu.sync_copy(pe_hbm, pe_v)

    def chunk(i, carry):
        row0 = base + i * _ROWS
        pltpu.sync_copy(x_hbm.at[pl.ds(row0, _ROWS)], idx_v)
        pltpu.async_copy(table_hbm.at[idx_v], rows_v, sem).wait()

        def add_row(r, c):
            for q in range(_DIM // _LANES):
                sl = pl.ds(q * _LANES, _LANES)
                rows_v[r, sl] = rows_v[r, sl] + pe_v[r, sl]
            return c

        lax.fori_loop(0, _ROWS, add_row, 0)
        pltpu.sync_copy(rows_v, out_hbm.at[pl.ds(row0, _ROWS)])
        return carry

    lax.fori_loop(0, _N_CHUNKS, chunk, 0)


@jax.jit
def _run(x_flat, pe, table):
    mesh = plsc.VectorSubcoreMesh(core_axis_name="c", subcore_axis_name="s")
    f = pl.kernel(
        _body,
        out_type=jax.ShapeDtypeStruct((_BATCH * _SEQ, _DIM), jnp.float32),
        mesh=mesh,
        scratch_types=[
            pltpu.VMEM((_ROWS,), jnp.int32),
            pltpu.VMEM((_ROWS, _DIM), jnp.float32),
            pltpu.VMEM((_ROWS, _DIM), jnp.float32),
            pltpu.SemaphoreType.DMA,
        ],
        compiler_params=pltpu.CompilerParams(use_tc_tiling_on_sc=False),
    )
    return f(x_flat, pe, table)


def kernel(x, table):
    out = _run(x.reshape(-1), _PE, table)
    return out.reshape(_BATCH, _SEQ, _DIM)
